# Initial kernel scaffold; baseline (speedup 1.0000x reference)
import jax
import jax.numpy as jnp
from jax.experimental import pallas as pl


def kernel(input_words, output_words, noise_words, embed_in, embed_out):
    raise NotImplementedError("write your pallas kernel here")


# speedup vs baseline: 7.4846x; 7.4846x over previous
import functools

import jax
import jax.numpy as jnp
from jax import lax
from jax.experimental import pallas as pl
from jax.experimental.pallas import tpu as pltpu
from jax.experimental.pallas import tpu_sc as plsc

VOCAB = 100000
EMBED = 128
BATCH = 16384
NOISE = 20

NCORES = 2
NSUB = 16
NW = NCORES * NSUB
BPW = BATCH // NW
CB = 32
NCHUNK = BPW // CB
NZROWS = CB * NOISE // 128
JPAD = 32
LANES = 16
KCH = EMBED // LANES


def _sc_scores(input_words, output_words, noise_flat, embed_in, embed_out):
  mesh = plsc.VectorSubcoreMesh(core_axis_name="c", subcore_axis_name="s")

  @functools.partial(
      pl.kernel,
      out_type=jax.ShapeDtypeStruct((BATCH, JPAD), jnp.float32),
      mesh=mesh,
      scratch_types=[
          pltpu.VMEM((CB,), jnp.int32),
          pltpu.VMEM((CB,), jnp.int32),
          pltpu.VMEM((BPW * NOISE,), jnp.int32),
          pltpu.VMEM((CB, EMBED), jnp.float32),
          pltpu.VMEM((CB, EMBED), jnp.float32),
          pltpu.VMEM((CB * NOISE, EMBED), jnp.float32),
          pltpu.VMEM((CB, JPAD), jnp.float32),
          pltpu.SemaphoreType.DMA,
      ],
  )
  def k(in_hbm, out_hbm, nz_hbm, ein_hbm, eout_hbm, scores_hbm,
        in_idx, out_idx, nz_idx, iv_rows, ov_rows, nv_rows, sc_v, sem):
    wid = lax.axis_index("s") * NCORES + lax.axis_index("c")
    base = wid * BPW
    lane = lax.broadcasted_iota(jnp.int32, (LANES,), 0)

    pltpu.sync_copy(nz_hbm.at[pl.ds(base * NOISE, BPW * NOISE)], nz_idx)

    def chunk_body(c, carry):
      b0 = base + c * CB
      c1 = pltpu.async_copy(in_hbm.at[pl.ds(b0, CB)], in_idx, sem)
      c2 = pltpu.async_copy(out_hbm.at[pl.ds(b0, CB)], out_idx, sem)
      c1.wait()
      c2.wait()
      g1 = pltpu.async_copy(ein_hbm.at[in_idx], iv_rows, sem)
      g2 = pltpu.async_copy(eout_hbm.at[out_idx], ov_rows, sem)
      gs = []
      for p in range(NZROWS):
        gs.append(pltpu.async_copy(
            eout_hbm.at[nz_idx.at[pl.ds(c * CB * NOISE + p * 128, 128)]],
            nv_rows.at[pl.ds(p * 128, 128)], sem))
      g1.wait()
      g2.wait()
      for g in gs:
        g.wait()

      gd = lax.GatherDimensionNumbers(
          offset_dims=(), collapsed_slice_dims=(0,), start_index_map=(0,))

      def xsum(acc):
        for sh in (8, 4, 2, 1):
          perm = lax.gather(
              acc, (lane ^ sh)[:, None], gd, slice_sizes=(1,),
              mode=lax.GatherScatterMode.PROMISE_IN_BOUNDS)
          acc = acc + perm
        return acc

      def b_body(bl, carry2):
        iv = [iv_rows[bl, pl.ds(LANES * kk, LANES)] for kk in range(KCH)]
        acc = iv[0] * ov_rows[bl, pl.ds(0, LANES)]
        for kk in range(1, KCH):
          acc = acc + iv[kk] * ov_rows[bl, pl.ds(LANES * kk, LANES)]
        s = xsum(acc)
        vec_a = jnp.where(lane == 0, s, 0.0)
        vec_b = jnp.zeros((LANES,), jnp.float32)

        def n_body(n, ab):
          a, b = ab
          r = bl * NOISE + n
          nacc = iv[0] * nv_rows[r, pl.ds(0, LANES)]
          for kk in range(1, KCH):
            nacc = nacc + iv[kk] * nv_rows[r, pl.ds(LANES * kk, LANES)]
          sn = -xsum(nacc)
          j = n + 1
          a = jnp.where(lane == j, sn, a)
          b = jnp.where(lane == j - LANES, sn, b)
          return a, b

        vec_a, vec_b = lax.fori_loop(0, NOISE, n_body, (vec_a, vec_b))
        sc_v[bl, pl.ds(0, LANES)] = vec_a
        sc_v[bl, pl.ds(LANES, LANES)] = vec_b
        return carry2

      lax.fori_loop(0, CB, b_body, 0)
      pltpu.sync_copy(sc_v, scores_hbm.at[pl.ds(b0, CB)])
      return carry

    lax.fori_loop(0, NCHUNK, chunk_body, 0)

  return k(input_words, output_words, noise_flat, embed_in, embed_out)


def _tc_loss(scores):
  def body(s_ref, o_ref):
    x = s_ref[...]
    col = lax.broadcasted_iota(jnp.int32, x.shape, 1)
    ls = jnp.minimum(x, 0.0) - jnp.log1p(jnp.exp(-jnp.abs(x)))
    ls = jnp.where(col < 1 + NOISE, ls, 0.0)
    o_ref[0, 0] = -jnp.sum(ls) / BATCH

  return pl.pallas_call(
      body,
      out_shape=jax.ShapeDtypeStruct((1, 1), jnp.float32),
      in_specs=[pl.BlockSpec(memory_space=pltpu.VMEM)],
      out_specs=pl.BlockSpec(memory_space=pltpu.SMEM),
  )(scores)


def kernel(input_words, output_words, noise_words, embed_in, embed_out):
  input_words = input_words.astype(jnp.int32)
  output_words = output_words.astype(jnp.int32)
  noise_flat = noise_words.astype(jnp.int32).reshape(BATCH * NOISE)
  scores = _sc_scores(input_words, output_words, noise_flat,
                      embed_in, embed_out)
  return _tc_loss(scores)[0, 0]


# speedup vs baseline: 11.3328x; 1.5141x over previous
import functools

import jax
import jax.numpy as jnp
from jax import lax
from jax.experimental import pallas as pl
from jax.experimental.pallas import tpu as pltpu
from jax.experimental.pallas import tpu_sc as plsc

VOCAB = 100000
EMBED = 128
BATCH = 16384
NOISE = 20

NCORES = 2
NSUB = 16
NW = NCORES * NSUB
BPW = BATCH // NW
CB = 16
NCHUNK = BPW // CB
JPAD = 32
LANES = 16
KCH = EMBED // LANES


def _sc_scores(input_words, output_words, noise_flat, embed_in, embed_out):
  mesh = plsc.VectorSubcoreMesh(core_axis_name="c", subcore_axis_name="s")

  @functools.partial(
      pl.kernel,
      out_type=jax.ShapeDtypeStruct((BATCH, JPAD), jnp.float32),
      mesh=mesh,
      scratch_types=[
          pltpu.VMEM((BPW,), jnp.int32),
          pltpu.VMEM((BPW,), jnp.int32),
          pltpu.VMEM((BPW * NOISE,), jnp.int32),
          pltpu.VMEM((2, CB, EMBED), jnp.float32),
          pltpu.VMEM((2, CB, EMBED), jnp.float32),
          pltpu.VMEM((2, CB * NOISE, EMBED), jnp.float32),
          pltpu.VMEM((CB, JPAD), jnp.float32),
          pltpu.SemaphoreType.DMA,
          pltpu.SemaphoreType.DMA,
      ],
  )
  def k(in_hbm, out_hbm, nz_hbm, ein_hbm, eout_hbm, scores_hbm,
        in_idx, out_idx, nz_idx, iv_rows, ov_rows, nv_rows, sc_v,
        sem0, sem1):
    wid = lax.axis_index("s") * NCORES + lax.axis_index("c")
    base = wid * BPW
    lane = lax.broadcasted_iota(jnp.int32, (LANES,), 0)

    pltpu.sync_copy(in_hbm.at[pl.ds(base, BPW)], in_idx)
    pltpu.sync_copy(out_hbm.at[pl.ds(base, BPW)], out_idx)
    pltpu.sync_copy(nz_hbm.at[pl.ds(base * NOISE, BPW * NOISE)], nz_idx)

    def transfers(c, buf, sem):
      o = c * CB
      ts = [
          pltpu.make_async_copy(
              ein_hbm.at[in_idx.at[pl.ds(o, CB)]], iv_rows.at[buf], sem),
          pltpu.make_async_copy(
              eout_hbm.at[out_idx.at[pl.ds(o, CB)]], ov_rows.at[buf], sem),
      ]
      for p in range(0, CB * NOISE, 128):
        n = min(128, CB * NOISE - p)
        ts.append(pltpu.make_async_copy(
            eout_hbm.at[nz_idx.at[pl.ds(o * NOISE + p, n)]],
            nv_rows.at[buf].at[pl.ds(p, n)], sem))
      return ts

    def fire(c, buf, sem):
      for t in transfers(c, buf, sem):
        t.start()

    def drain(c, buf, sem):
      for t in transfers(c, buf, sem):
        t.wait()

    gd = lax.GatherDimensionNumbers(
        offset_dims=(), collapsed_slice_dims=(0,), start_index_map=(0,))

    def xsum(acc):
      for sh in (8, 4, 2, 1):
        perm = lax.gather(
            acc, (lane ^ sh)[:, None], gd, slice_sizes=(1,),
            mode=lax.GatherScatterMode.PROMISE_IN_BOUNDS)
        acc = acc + perm
      return acc

    def compute(c, buf):
      def b_body(bl, carry2):
        iv = [iv_rows[buf, bl, pl.ds(LANES * kk, LANES)] for kk in range(KCH)]
        acc = iv[0] * ov_rows[buf, bl, pl.ds(0, LANES)]
        for kk in range(1, KCH):
          acc = acc + iv[kk] * ov_rows[buf, bl, pl.ds(LANES * kk, LANES)]
        s = xsum(acc)
        vec_a = jnp.where(lane == 0, s, 0.0)
        vec_b = jnp.zeros((LANES,), jnp.float32)

        def n_body(n, ab):
          a, b = ab
          r = bl * NOISE + n
          nacc = iv[0] * nv_rows[buf, r, pl.ds(0, LANES)]
          for kk in range(1, KCH):
            nacc = nacc + iv[kk] * nv_rows[buf, r, pl.ds(LANES * kk, LANES)]
          sn = -xsum(nacc)
          j = n + 1
          a = jnp.where(lane == j, sn, a)
          b = jnp.where(lane == j - LANES, sn, b)
          return a, b

        vec_a, vec_b = lax.fori_loop(0, NOISE, n_body, (vec_a, vec_b))
        sc_v[bl, pl.ds(0, LANES)] = vec_a
        sc_v[bl, pl.ds(LANES, LANES)] = vec_b
        return carry2

      lax.fori_loop(0, CB, b_body, 0)
      pltpu.sync_copy(sc_v, scores_hbm.at[pl.ds(base + c * CB, CB)])

    fire(0, 0, sem0)

    def pair_body(i, carry):
      c0 = 2 * i
      fire(c0 + 1, 1, sem1)
      drain(c0, 0, sem0)
      compute(c0, 0)

      @pl.when(c0 + 2 < NCHUNK)
      def _():
        fire(c0 + 2, 0, sem0)

      drain(c0 + 1, 1, sem1)
      compute(c0 + 1, 1)
      return carry

    lax.fori_loop(0, NCHUNK // 2, pair_body, 0)

  return k(input_words, output_words, noise_flat, embed_in, embed_out)


def _tc_loss(scores):
  def body(s_ref, o_ref):
    x = s_ref[...]
    col = lax.broadcasted_iota(jnp.int32, x.shape, 1)
    ls = jnp.minimum(x, 0.0) - jnp.log1p(jnp.exp(-jnp.abs(x)))
    ls = jnp.where(col < 1 + NOISE, ls, 0.0)
    o_ref[0, 0] = -jnp.sum(ls) / BATCH

  return pl.pallas_call(
      body,
      out_shape=jax.ShapeDtypeStruct((1, 1), jnp.float32),
      in_specs=[pl.BlockSpec(memory_space=pltpu.VMEM)],
      out_specs=pl.BlockSpec(memory_space=pltpu.SMEM),
  )(scores)


def kernel(input_words, output_words, noise_words, embed_in, embed_out):
  input_words = input_words.astype(jnp.int32)
  output_words = output_words.astype(jnp.int32)
  noise_flat = noise_words.astype(jnp.int32).reshape(BATCH * NOISE)
  scores = _sc_scores(input_words, output_words, noise_flat,
                      embed_in, embed_out)
  return _tc_loss(scores)[0, 0]


# speedup vs baseline: 11.7471x; 1.0366x over previous
import functools

import jax
import jax.numpy as jnp
from jax import lax
from jax.experimental import pallas as pl
from jax.experimental.pallas import tpu as pltpu
from jax.experimental.pallas import tpu_sc as plsc

VOCAB = 100000
EMBED = 128
BATCH = 16384
NOISE = 20

NCORES = 2
NSUB = 16
NW = NCORES * NSUB
BPW = BATCH // NW
CB = 16
NCHUNK = BPW // CB
JPAD = 32
LANES = 16
KCH = EMBED // LANES


def _sc_scores(input_words, output_words, noise_flat, embed_in, embed_out):
  mesh = plsc.VectorSubcoreMesh(core_axis_name="c", subcore_axis_name="s")

  @functools.partial(
      pl.kernel,
      out_type=jax.ShapeDtypeStruct((BATCH, JPAD), jnp.float32),
      mesh=mesh,
      scratch_types=[
          pltpu.VMEM((BPW,), jnp.int32),
          pltpu.VMEM((BPW,), jnp.int32),
          pltpu.VMEM((BPW * NOISE,), jnp.int32),
          pltpu.VMEM((2, CB, EMBED), jnp.float32),
          pltpu.VMEM((2, CB, EMBED), jnp.float32),
          pltpu.VMEM((2, CB * NOISE, EMBED), jnp.float32),
          pltpu.VMEM((CB, JPAD), jnp.float32),
          pltpu.SemaphoreType.DMA,
          pltpu.SemaphoreType.DMA,
      ],
  )
  def k(in_hbm, out_hbm, nz_hbm, ein_hbm, eout_hbm, scores_hbm,
        in_idx, out_idx, nz_idx, iv_rows, ov_rows, nv_rows, sc_v,
        sem0, sem1):
    wid = lax.axis_index("s") * NCORES + lax.axis_index("c")
    base = wid * BPW
    lane = lax.broadcasted_iota(jnp.int32, (LANES,), 0)

    pltpu.sync_copy(in_hbm.at[pl.ds(base, BPW)], in_idx)
    pltpu.sync_copy(out_hbm.at[pl.ds(base, BPW)], out_idx)
    pltpu.sync_copy(nz_hbm.at[pl.ds(base * NOISE, BPW * NOISE)], nz_idx)

    def transfers(c, buf, sem):
      o = c * CB
      ts = [
          pltpu.make_async_copy(
              ein_hbm.at[in_idx.at[pl.ds(o, CB)]], iv_rows.at[buf], sem),
          pltpu.make_async_copy(
              eout_hbm.at[out_idx.at[pl.ds(o, CB)]], ov_rows.at[buf], sem),
      ]
      for p in range(0, CB * NOISE, 128):
        n = min(128, CB * NOISE - p)
        ts.append(pltpu.make_async_copy(
            eout_hbm.at[nz_idx.at[pl.ds(o * NOISE + p, n)]],
            nv_rows.at[buf].at[pl.ds(p, n)], sem))
      return ts

    def fire(c, buf, sem):
      for t in transfers(c, buf, sem):
        t.start()

    def drain(c, buf, sem):
      for t in transfers(c, buf, sem):
        t.wait()

    gd = lax.GatherDimensionNumbers(
        offset_dims=(), collapsed_slice_dims=(0,), start_index_map=(0,))

    def xsum(acc):
      for sh in (8, 4, 2, 1):
        perm = lax.gather(
            acc, (lane ^ sh)[:, None], gd, slice_sizes=(1,),
            mode=lax.GatherScatterMode.PROMISE_IN_BOUNDS)
        acc = acc + perm
      return acc

    def compute(c, buf):
      def b_body(bl, carry2):
        iv = [iv_rows[buf, bl, pl.ds(LANES * kk, LANES)] for kk in range(KCH)]
        acc = iv[0] * ov_rows[buf, bl, pl.ds(0, LANES)]
        for kk in range(1, KCH):
          acc = acc + iv[kk] * ov_rows[buf, bl, pl.ds(LANES * kk, LANES)]
        s = xsum(acc)
        vec_a = jnp.where(lane == 0, s, 0.0)
        vec_b = jnp.zeros((LANES,), jnp.float32)

        for n in range(NOISE):
          r = bl * NOISE + n
          nacc = iv[0] * nv_rows[buf, r, pl.ds(0, LANES)]
          for kk in range(1, KCH):
            nacc = nacc + iv[kk] * nv_rows[buf, r, pl.ds(LANES * kk, LANES)]
          sn = -xsum(nacc)
          j = n + 1
          if j < LANES:
            vec_a = jnp.where(lane == j, sn, vec_a)
          else:
            vec_b = jnp.where(lane == j - LANES, sn, vec_b)
        sc_v[bl, pl.ds(0, LANES)] = vec_a
        sc_v[bl, pl.ds(LANES, LANES)] = vec_b
        return carry2

      lax.fori_loop(0, CB, b_body, 0)
      pltpu.sync_copy(sc_v, scores_hbm.at[pl.ds(base + c * CB, CB)])

    fire(0, 0, sem0)

    def pair_body(i, carry):
      c0 = 2 * i
      fire(c0 + 1, 1, sem1)
      drain(c0, 0, sem0)
      compute(c0, 0)

      @pl.when(c0 + 2 < NCHUNK)
      def _():
        fire(c0 + 2, 0, sem0)

      drain(c0 + 1, 1, sem1)
      compute(c0 + 1, 1)
      return carry

    lax.fori_loop(0, NCHUNK // 2, pair_body, 0)

  return k(input_words, output_words, noise_flat, embed_in, embed_out)


def _tc_loss(scores):
  def body(s_ref, o_ref):
    x = s_ref[...]
    col = lax.broadcasted_iota(jnp.int32, x.shape, 1)
    ls = jnp.minimum(x, 0.0) - jnp.log1p(jnp.exp(-jnp.abs(x)))
    ls = jnp.where(col < 1 + NOISE, ls, 0.0)
    o_ref[0, 0] = -jnp.sum(ls) / BATCH

  return pl.pallas_call(
      body,
      out_shape=jax.ShapeDtypeStruct((1, 1), jnp.float32),
      in_specs=[pl.BlockSpec(memory_space=pltpu.VMEM)],
      out_specs=pl.BlockSpec(memory_space=pltpu.SMEM),
  )(scores)


def kernel(input_words, output_words, noise_words, embed_in, embed_out):
  input_words = input_words.astype(jnp.int32)
  output_words = output_words.astype(jnp.int32)
  noise_flat = noise_words.astype(jnp.int32).reshape(BATCH * NOISE)
  scores = _sc_scores(input_words, output_words, noise_flat,
                      embed_in, embed_out)
  return _tc_loss(scores)[0, 0]
